# baseline (speedup 1.0000x reference)
import functools
import math

import jax
import jax.numpy as jnp
from jax import lax
from jax.experimental import pallas as pl
from jax.experimental.pallas import tpu as pltpu
from jax.experimental.pallas import tpu_sc as plsc


def _gelu_exact(x):
    return 0.5 * x * (1.0 + lax.erf(x / math.sqrt(2.0)))


def _swar_popcount(x):
    x = x - (jnp.right_shift(x, 1) & 0x55555555)
    x = (x & 0x33333333) + (jnp.right_shift(x, 2) & 0x33333333)
    x = (x + jnp.right_shift(x, 4)) & 0x0F0F0F0F
    return jnp.right_shift(x * 0x01010101, 24) & 0x3F


def _sc_stage(tables, sigma, meta):
    B, V, N = tables.shape
    info = plsc.get_sparse_core_info()
    NC, NS, L = info.num_cores, info.num_subcores, info.num_lanes
    NW = NC * NS
    RPW = B // NW
    CH = N // L
    mesh = plsc.VectorSubcoreMesh(core_axis_name="c", subcore_axis_name="s")

    @functools.partial(
        pl.kernel,
        mesh=mesh,
        compiler_params=pltpu.CompilerParams(needs_layout_passes=False),
        out_type=jax.ShapeDtypeStruct((B, 64), jnp.float32),
        scratch_types=[
            pltpu.VMEM((V, N), jnp.int32),
            pltpu.VMEM((V, N), jnp.int32),
            pltpu.VMEM((N,), jnp.int32),
            pltpu.VMEM((N,), jnp.int32),
            pltpu.VMEM((RPW * 16,), jnp.int32),
            pltpu.VMEM((32,), jnp.float32),
            pltpu.VMEM((V * 32,), jnp.float32),
            pltpu.VMEM((64,), jnp.float32),
            pltpu.SemaphoreType.DMA,
            pltpu.SemaphoreType.DMA,
        ],
    )
    def sc(tables_hbm, sigma_hbm, meta_hbm, out_hbm,
           row0_v, row1_v, sig0_v, sig1_v, meta_v, hist_v, seen_v, ost_v,
           sem0, sem1):
        wid = lax.axis_index("s") * NC + lax.axis_index("c")
        base_row = wid * RPW
        pltpu.sync_copy(meta_hbm.at[pl.ds(base_row * 16, RPW * 16)], meta_v)
        v_iota = lax.iota(jnp.int32, L)
        ones16f = jnp.ones((L,), jnp.float32)
        zeros16f = jnp.zeros((L,), jnp.float32)

        slots = ((row0_v, sig0_v, sem0), (row1_v, sig1_v, sem1))

        def start_fetch(slot, b):
            row_v, sig_v, sem = slot
            pltpu.async_copy(tables_hbm.at[b], row_v, sem)
            pltpu.async_copy(sigma_hbm.at[b], sig_v, sem)

        def wait_fetch(slot, b):
            row_v, sig_v, sem = slot
            pltpu.make_async_copy(tables_hbm.at[b], row_v, sem).wait()
            pltpu.make_async_copy(sigma_hbm.at[b], sig_v, sem).wait()

        def process_row(row_v, sig_v, r):
            b = base_row + r
            mvec = meta_v[pl.ds(r * 16, 16)]

            def mget(j):
                return jnp.sum(jnp.where(v_iota == j, mvec, 0))

            req = jnp.where(v_iota == 0, mget(0), -1)
            clash = jnp.zeros((L,), jnp.bool_)
            for i in range(4):
                a = mget(1 + i)
                rr = mget(5 + i)
                hit = v_iota == a
                clash = clash | (hit & (req >= 0) & (req != rr))
                req = jnp.where(hit & (req < 0), rr, req)
            n_clash = plsc.all_reduce_population_count(clash)
            zo = jnp.where(n_clash > 0, zeros16f, ones16f)

            req_s = [jnp.zeros((L,), jnp.int32)
                     + jnp.sum(jnp.where(v_iota == v, req, 0))
                     for v in range(V)]
            unc = [rs < 0 for rs in req_s]

            hist_v[pl.ds(0, L)] = zeros16f
            hist_v[pl.ds(L, L)] = zeros16f
            for v in range(V):
                seen_v[pl.ds(v * 32, L)] = zeros16f
                seen_v[pl.ds(v * 32 + L, L)] = zeros16f

            @plsc.parallel_loop(0, CH, unroll=4)
            def chunk_body(n):
                off = n * L
                tvs = [row_v[v, pl.ds(off, L)] for v in range(V)]
                m = unc[0] | (tvs[0] == req_s[0])
                for v in range(1, V):
                    m = m & (unc[v] | (tvs[v] == req_s[v]))
                @pl.when(jnp.any(m))
                def _():
                    sv = sig_v[pl.ds(off, L)]
                    plsc.addupdate_scatter(hist_v, [sv], ones16f, mask=m)
                    for v in range(V):
                        plsc.store_scatter(seen_v, [tvs[v] + (v * 32)],
                                           ones16f, mask=m)

            ost_v[pl.ds(0, L)] = hist_v[pl.ds(0, L)] * zo
            ost_v[pl.ds(L, L)] = hist_v[pl.ds(L, L)] * zo
            seg2 = jnp.zeros((L,), jnp.float32)
            for v in range(V):
                sm = jnp.sum(seen_v[pl.ds(v * 32, L)]
                             + seen_v[pl.ds(v * 32 + L, L)])
                seg2 = seg2 + jnp.where(v_iota == v, sm, 0.0)
            ost_v[pl.ds(2 * L, L)] = seg2 * zo
            ost_v[pl.ds(3 * L, L)] = zeros16f
            pltpu.sync_copy(ost_v, out_hbm.at[b])

        start_fetch(slots[0], base_row)

        def pair_body(p, carry):
            r0 = 2 * p
            b0 = base_row + r0
            start_fetch(slots[1], b0 + 1)
            wait_fetch(slots[0], b0)
            process_row(row0_v, sig0_v, r0)

            @pl.when(p < RPW // 2 - 1)
            def _():
                start_fetch(slots[0], b0 + 2)

            wait_fetch(slots[1], b0 + 1)
            process_row(row1_v, sig1_v, r0 + 1)
            return carry

        lax.fori_loop(0, RPW // 2, pair_body, 0)

    return sc(tables, sigma, meta)


def _mlp_body(ft_ref, wz1_ref, bz1_ref, wz2_ref, bz2_ref,
              wy1_ref, by1_ref, wy2_ref, by2_ref, out_ref):
    ft = ft_ref[...]
    BB = ft.shape[0]
    V = 8
    hist = ft[:, :32]
    uniq = ft[:, 32:32 + V]
    cnt = jnp.sum(hist, axis=1, keepdims=True)

    zden = jnp.maximum(cnt, 1.0)
    p_sig = hist / zden
    mass = jnp.where(cnt > 0, 1.0 / zden, 0.0)

    c_iota = lax.broadcasted_iota(jnp.int32, (BB, 32), 1)
    pc = jnp.maximum(p_sig, 1e-9)
    ent = -jnp.sum(pc * jnp.log(pc), axis=1, keepdims=True)

    mx = jnp.max(p_sig, axis=1, keepdims=True)
    idx1 = jnp.min(jnp.where(p_sig >= mx, c_iota, 32), axis=1, keepdims=True)
    second = jnp.max(jnp.where(c_iota == idx1, -jnp.inf, p_sig),
                     axis=1, keepdims=True)

    feat = jnp.concatenate([p_sig, uniq, ent, mx, second, mass], axis=1)
    h1 = _gelu_exact(
        jnp.dot(feat, wz1_ref[...], preferred_element_type=jnp.float32)
        + bz1_ref[...])
    zl = jnp.dot(h1, wz2_ref[...], preferred_element_type=jnp.float32) + bz2_ref[...]

    v8 = lax.broadcasted_iota(jnp.int32, (BB, V), 1)
    s = jnp.exp(zl - jnp.max(zl, axis=1, keepdims=True))
    zs = s / jnp.sum(s, axis=1, keepdims=True)
    mz = jnp.max(zs, axis=1, keepdims=True)
    iz = jnp.min(jnp.where(zs >= mz, v8, V), axis=1, keepdims=True)
    zoh = (v8 == iz).astype(jnp.float32)

    feat2 = jnp.concatenate([p_sig, zoh], axis=1)
    h2 = _gelu_exact(
        jnp.dot(feat2, wy1_ref[...], preferred_element_type=jnp.float32)
        + by1_ref[...])
    out_ref[...] = (
        jnp.dot(h2, wy2_ref[...], preferred_element_type=jnp.float32)
        + by2_ref[...])


def kernel(tables, sigma, base_obs, actions, responses, t,
           W_z1, b_z1, W_z2, b_z2, W_y1, b_y1, W_y2, b_y2):
    B, V, N = tables.shape
    T = actions.shape[1]
    C = W_y2.shape[1]

    act_eff = jnp.where(jnp.arange(T)[None, :] < t,
                        jnp.clip(actions, 0, V - 1), V)
    meta = jnp.concatenate(
        [base_obs.reshape(B, 1), act_eff, responses,
         jnp.zeros((B, 16 - 1 - 2 * T), jnp.int32)], axis=1)

    feats = _sc_stage(tables, sigma, meta.reshape(B * 16))

    full = lambda shape: pl.BlockSpec(shape, lambda *_: (0,) * len(shape))
    out = pl.pallas_call(
        _mlp_body,
        in_specs=[
            full((B, 64)),
            full(W_z1.shape),
            full((1, b_z1.shape[0])),
            full(W_z2.shape),
            full((1, b_z2.shape[0])),
            full(W_y1.shape),
            full((1, b_y1.shape[0])),
            full(W_y2.shape),
            full((1, b_y2.shape[0])),
        ],
        out_specs=full((B, C)),
        out_shape=jax.ShapeDtypeStruct((B, C), jnp.float32),
    )(feats,
      W_z1, b_z1.reshape(1, -1), W_z2, b_z2.reshape(1, -1),
      W_y1, b_y1.reshape(1, -1), W_y2, b_y2.reshape(1, -1))
    return out


# speedup vs baseline: 1.4919x; 1.4919x over previous
import functools
import math

import jax
import jax.numpy as jnp
from jax import lax
from jax.experimental import pallas as pl
from jax.experimental.pallas import tpu as pltpu
from jax.experimental.pallas import tpu_sc as plsc


def _gelu_exact(x):
    return 0.5 * x * (1.0 + lax.erf(x / math.sqrt(2.0)))


def _swar_popcount(x):
    x = x - (jnp.right_shift(x, 1) & 0x55555555)
    x = (x & 0x33333333) + (jnp.right_shift(x, 2) & 0x33333333)
    x = (x + jnp.right_shift(x, 4)) & 0x0F0F0F0F
    return jnp.right_shift(x * 0x01010101, 24) & 0x3F


def _sc_stage(tables, sigma, meta):
    B, V, N = tables.shape
    info = plsc.get_sparse_core_info()
    NC, NS, L = info.num_cores, info.num_subcores, info.num_lanes
    NW = NC * NS
    RPW = B // NW
    CH = N // L
    mesh = plsc.VectorSubcoreMesh(core_axis_name="c", subcore_axis_name="s")

    @functools.partial(
        pl.kernel,
        mesh=mesh,
        compiler_params=pltpu.CompilerParams(needs_layout_passes=False),
        out_type=jax.ShapeDtypeStruct((B, 64), jnp.float32),
        scratch_types=[
            pltpu.VMEM((V, N), jnp.int32),
            pltpu.VMEM((V, N), jnp.int32),
            pltpu.VMEM((N,), jnp.int32),
            pltpu.VMEM((N,), jnp.int32),
            pltpu.VMEM((RPW * 16,), jnp.int32),
            pltpu.VMEM((32,), jnp.float32),
            pltpu.VMEM((V * 32,), jnp.float32),
            pltpu.VMEM((64,), jnp.float32),
            pltpu.SemaphoreType.DMA,
            pltpu.SemaphoreType.DMA,
        ],
    )
    def sc(tables_hbm, sigma_hbm, meta_hbm, out_hbm,
           row0_v, row1_v, sig0_v, sig1_v, meta_v, hist_v, seen_v, ost_v,
           sem0, sem1):
        wid = lax.axis_index("s") * NC + lax.axis_index("c")
        base_row = wid * RPW
        pltpu.sync_copy(meta_hbm.at[pl.ds(base_row * 16, RPW * 16)], meta_v)
        v_iota = lax.iota(jnp.int32, L)
        ones16f = jnp.ones((L,), jnp.float32)
        zeros16f = jnp.zeros((L,), jnp.float32)

        slots = ((row0_v, sig0_v, sem0), (row1_v, sig1_v, sem1))

        def start_fetch(slot, b):
            row_v, sig_v, sem = slot
            pltpu.async_copy(tables_hbm.at[b], row_v, sem)
            pltpu.async_copy(sigma_hbm.at[b], sig_v, sem)

        def wait_fetch(slot, b):
            row_v, sig_v, sem = slot
            pltpu.make_async_copy(tables_hbm.at[b], row_v, sem).wait()
            pltpu.make_async_copy(sigma_hbm.at[b], sig_v, sem).wait()

        def process_row(row_v, sig_v, r):
            b = base_row + r
            mvec = meta_v[pl.ds(r * 16, 16)]

            def mget(j):
                return jnp.sum(jnp.where(v_iota == j, mvec, 0))

            req = jnp.where(v_iota == 0, mget(0), -1)
            clash = jnp.zeros((L,), jnp.bool_)
            for i in range(4):
                a = mget(1 + i)
                rr = mget(5 + i)
                hit = v_iota == a
                clash = clash | (hit & (req >= 0) & (req != rr))
                req = jnp.where(hit & (req < 0), rr, req)
            n_clash = plsc.all_reduce_population_count(clash)
            zo = jnp.where(n_clash > 0, zeros16f, ones16f)

            req_s = [jnp.zeros((L,), jnp.int32)
                     + jnp.sum(jnp.where(v_iota == v, req, 0))
                     for v in range(V)]
            unc = [rs < 0 for rs in req_s]

            hist_v[pl.ds(0, L)] = zeros16f
            hist_v[pl.ds(L, L)] = zeros16f
            for v in range(V):
                seen_v[pl.ds(v * 32, L)] = zeros16f
                seen_v[pl.ds(v * 32 + L, L)] = zeros16f

            @plsc.parallel_loop(0, CH, unroll=4)
            def chunk_body(n):
                off = n * L
                tvs = [row_v[v, pl.ds(off, L)] for v in range(V)]
                m = unc[0] | (tvs[0] == req_s[0])
                for v in range(1, V):
                    m = m & (unc[v] | (tvs[v] == req_s[v]))
                sv = sig_v[pl.ds(off, L)]
                plsc.addupdate_scatter(hist_v, [sv], ones16f, mask=m)
                for v in range(V):
                    plsc.store_scatter(seen_v, [tvs[v] + (v * 32)],
                                       ones16f, mask=m)

            ost_v[pl.ds(0, L)] = hist_v[pl.ds(0, L)] * zo
            ost_v[pl.ds(L, L)] = hist_v[pl.ds(L, L)] * zo
            seg2 = jnp.zeros((L,), jnp.float32)
            for v in range(V):
                sm = jnp.sum(seen_v[pl.ds(v * 32, L)]
                             + seen_v[pl.ds(v * 32 + L, L)])
                seg2 = seg2 + jnp.where(v_iota == v, sm, 0.0)
            ost_v[pl.ds(2 * L, L)] = seg2 * zo
            ost_v[pl.ds(3 * L, L)] = zeros16f
            pltpu.sync_copy(ost_v, out_hbm.at[b])

        start_fetch(slots[0], base_row)

        def pair_body(p, carry):
            r0 = 2 * p
            b0 = base_row + r0
            start_fetch(slots[1], b0 + 1)
            wait_fetch(slots[0], b0)
            process_row(row0_v, sig0_v, r0)

            @pl.when(p < RPW // 2 - 1)
            def _():
                start_fetch(slots[0], b0 + 2)

            wait_fetch(slots[1], b0 + 1)
            process_row(row1_v, sig1_v, r0 + 1)
            return carry

        lax.fori_loop(0, RPW // 2, pair_body, 0)

    return sc(tables, sigma, meta)


def _mlp_body(ft_ref, wz1_ref, bz1_ref, wz2_ref, bz2_ref,
              wy1_ref, by1_ref, wy2_ref, by2_ref, out_ref):
    ft = ft_ref[...]
    BB = ft.shape[0]
    V = 8
    hist = ft[:, :32]
    uniq = ft[:, 32:32 + V]
    cnt = jnp.sum(hist, axis=1, keepdims=True)

    zden = jnp.maximum(cnt, 1.0)
    p_sig = hist / zden
    mass = jnp.where(cnt > 0, 1.0 / zden, 0.0)

    c_iota = lax.broadcasted_iota(jnp.int32, (BB, 32), 1)
    pc = jnp.maximum(p_sig, 1e-9)
    ent = -jnp.sum(pc * jnp.log(pc), axis=1, keepdims=True)

    mx = jnp.max(p_sig, axis=1, keepdims=True)
    idx1 = jnp.min(jnp.where(p_sig >= mx, c_iota, 32), axis=1, keepdims=True)
    second = jnp.max(jnp.where(c_iota == idx1, -jnp.inf, p_sig),
                     axis=1, keepdims=True)

    feat = jnp.concatenate([p_sig, uniq, ent, mx, second, mass], axis=1)
    h1 = _gelu_exact(
        jnp.dot(feat, wz1_ref[...], preferred_element_type=jnp.float32)
        + bz1_ref[...])
    zl = jnp.dot(h1, wz2_ref[...], preferred_element_type=jnp.float32) + bz2_ref[...]

    v8 = lax.broadcasted_iota(jnp.int32, (BB, V), 1)
    s = jnp.exp(zl - jnp.max(zl, axis=1, keepdims=True))
    zs = s / jnp.sum(s, axis=1, keepdims=True)
    mz = jnp.max(zs, axis=1, keepdims=True)
    iz = jnp.min(jnp.where(zs >= mz, v8, V), axis=1, keepdims=True)
    zoh = (v8 == iz).astype(jnp.float32)

    feat2 = jnp.concatenate([p_sig, zoh], axis=1)
    h2 = _gelu_exact(
        jnp.dot(feat2, wy1_ref[...], preferred_element_type=jnp.float32)
        + by1_ref[...])
    out_ref[...] = (
        jnp.dot(h2, wy2_ref[...], preferred_element_type=jnp.float32)
        + by2_ref[...])


def kernel(tables, sigma, base_obs, actions, responses, t,
           W_z1, b_z1, W_z2, b_z2, W_y1, b_y1, W_y2, b_y2):
    B, V, N = tables.shape
    T = actions.shape[1]
    C = W_y2.shape[1]

    act_eff = jnp.where(jnp.arange(T)[None, :] < t,
                        jnp.clip(actions, 0, V - 1), V)
    meta = jnp.concatenate(
        [base_obs.reshape(B, 1), act_eff, responses,
         jnp.zeros((B, 16 - 1 - 2 * T), jnp.int32)], axis=1)

    feats = _sc_stage(tables, sigma, meta.reshape(B * 16))

    full = lambda shape: pl.BlockSpec(shape, lambda *_: (0,) * len(shape))
    out = pl.pallas_call(
        _mlp_body,
        in_specs=[
            full((B, 64)),
            full(W_z1.shape),
            full((1, b_z1.shape[0])),
            full(W_z2.shape),
            full((1, b_z2.shape[0])),
            full(W_y1.shape),
            full((1, b_y1.shape[0])),
            full(W_y2.shape),
            full((1, b_y2.shape[0])),
        ],
        out_specs=full((B, C)),
        out_shape=jax.ShapeDtypeStruct((B, C), jnp.float32),
    )(feats,
      W_z1, b_z1.reshape(1, -1), W_z2, b_z2.reshape(1, -1),
      W_y1, b_y1.reshape(1, -1), W_y2, b_y2.reshape(1, -1))
    return out
